# baseline (speedup 1.0000x reference)
import functools

import jax
import jax.numpy as jnp
from jax import lax
from jax.experimental import pallas as pl
from jax.experimental.pallas import tpu as pltpu
from jax.experimental.pallas import tpu_sc as plsc

K = 2
TAU = 0.01
MAX_ITER = 100
TOL = 1e-3

N_ROWS = 16384
N_COLS = 8
WORKERS = 32
RPW = N_ROWS // WORKERS
VPW = RPW * N_COLS
RCHUNKS = VPW // 16
TCHUNKS = RPW // 16


def _take(x, idx):
    return x.at[idx].get(mode="promise_in_bounds")


def _seg_sum(x, iota):
    x = x + _take(x, iota ^ 1)
    x = x + _take(x, iota ^ 2)
    x = x + _take(x, iota ^ 4)
    return x


def _sigmoid(x):
    return 1.0 / (1.0 + jnp.exp(-x))


def _third_largest_seg(x, iota, p):
    def layer(x, q):
        partner = _take(x, iota - p + q)
        return jnp.where(p < q, jnp.minimum(x, partner),
                         jnp.maximum(x, partner))

    x = layer(x, p ^ 1)
    x = layer(x, p ^ 2)
    sel = (p == 1) | (p == 2) | (p == 5) | (p == 6)
    x = layer(x, jnp.where(sel, p ^ 3, p))
    x = layer(x, p ^ 4)
    x = layer(x, jnp.where((p >= 2) & (p <= 5), p ^ 6, p))
    x = layer(x, jnp.where((p >= 1) & (p <= 6), ((p - 1) ^ 1) + 1, p))
    return _take(x, iota - p + 5)


def _worker_id():
    return lax.axis_index("s") * 2 + lax.axis_index("c")


def _sc_init(s_hbm, theta_hbm, raw, tloc):
    wid = _worker_id()
    iota = jnp.arange(16, dtype=jnp.int32)
    p = iota & 7
    pltpu.sync_copy(s_hbm.at[pl.ds(VPW * wid, VPW)], raw)

    lane_pair = iota >> 1
    odd = (iota & 1) << 3

    def rows16(c, _):
        acc = jnp.zeros(16, jnp.float32)
        for k in range(8):
            t = _third_largest_seg(raw[pl.ds(16 * (8 * c + k), 16)], iota, p)
            acc = jnp.where(lane_pair == k, _take(t, odd), acc)
        tloc[pl.ds(16 * c, 16)] = acc
        return 0
    lax.fori_loop(0, TCHUNKS, rows16, 0)
    pltpu.sync_copy(tloc, theta_hbm.at[pl.ds(RPW * wid, RPW)])


def _sc_pass(s_hbm, th_hbm, ratio_hbm, parts_hbm, raw, tloc, rloc, facc):
    wid = _worker_id()
    iota = jnp.arange(16, dtype=jnp.int32)
    pltpu.sync_copy(s_hbm.at[pl.ds(VPW * wid, VPW)], raw)
    pltpu.sync_copy(th_hbm.at[pl.ds(RPW * wid, RPW)], tloc)
    facc[...] = jnp.zeros(16, jnp.float32)

    lane_pair = iota >> 1
    odd = (iota & 1) << 3
    exp_idx = iota >> 3

    def rows16(c, _):
        th16 = tloc[pl.ds(16 * c, 16)]
        acc = jnp.zeros(16, jnp.float32)
        for k in range(8):
            th = _take(th16, exp_idx + 2 * k)
            sg = _sigmoid((raw[pl.ds(16 * (8 * c + k), 16)] - th) / TAU)
            facc[...] = facc[...] + sg
            f = _seg_sum(sg, iota) - K
            df = -(1.0 / TAU) * _seg_sum(sg * (1.0 - sg), iota)
            step = f / df
            acc = jnp.where(lane_pair == k, _take(step, odd), acc)
        rloc[pl.ds(16 * c, 16)] = acc
        return 0
    lax.fori_loop(0, TCHUNKS, rows16, 0)

    pltpu.sync_copy(rloc, ratio_hbm.at[pl.ds(RPW * wid, RPW)])
    pltpu.sync_copy(facc, parts_hbm.at[pl.ds(16 * wid, 16)])


def _sc_final(s_hbm, th_hbm, g_hbm, raw, tloc, gloc):
    wid = _worker_id()
    iota = jnp.arange(16, dtype=jnp.int32)
    pltpu.sync_copy(s_hbm.at[pl.ds(VPW * wid, VPW)], raw)
    pltpu.sync_copy(th_hbm.at[pl.ds(RPW * wid, RPW)], tloc)

    exp_idx = iota >> 3

    def rows16(c, _):
        th16 = tloc[pl.ds(16 * c, 16)]
        for k in range(8):
            th = _take(th16, exp_idx + 2 * k)
            off = 16 * (8 * c + k)
            gloc[pl.ds(off, 16)] = _sigmoid((raw[pl.ds(off, 16)] - th) / TAU)
        return 0
    lax.fori_loop(0, TCHUNKS, rows16, 0)
    pltpu.sync_copy(gloc, g_hbm.at[pl.ds(VPW * wid, VPW)])


def _mesh():
    return plsc.VectorSubcoreMesh(core_axis_name="c", subcore_axis_name="s")


_init_call = functools.partial(
    pl.kernel,
    mesh=_mesh(),
    out_type=jax.ShapeDtypeStruct((N_ROWS,), jnp.float32),
    scratch_types=[
        pltpu.VMEM((VPW,), jnp.float32),
        pltpu.VMEM((RPW,), jnp.float32),
    ],
)(_sc_init)

_pass_call = functools.partial(
    pl.kernel,
    mesh=_mesh(),
    out_type=(
        jax.ShapeDtypeStruct((N_ROWS,), jnp.float32),
        jax.ShapeDtypeStruct((16 * WORKERS,), jnp.float32),
    ),
    scratch_types=[
        pltpu.VMEM((VPW,), jnp.float32),
        pltpu.VMEM((RPW,), jnp.float32),
        pltpu.VMEM((RPW,), jnp.float32),
        pltpu.VMEM((16,), jnp.float32),
    ],
)(_sc_pass)

_final_call = functools.partial(
    pl.kernel,
    mesh=_mesh(),
    out_type=jax.ShapeDtypeStruct((N_ROWS * N_COLS,), jnp.float32),
    scratch_types=[
        pltpu.VMEM((VPW,), jnp.float32),
        pltpu.VMEM((RPW,), jnp.float32),
        pltpu.VMEM((VPW,), jnp.float32),
    ],
)(_sc_final)


@jax.jit
def kernel(s):
    s_flat = s.reshape(N_ROWS * N_COLS)
    theta0 = _init_call(s_flat)

    def body(carry):
        theta, i, done = carry
        ratio, parts = _pass_call(s_flat, theta)
        mean_f = (jnp.sum(parts) - jnp.float32(K * N_ROWS)) / N_ROWS
        new_done = mean_f < TOL
        theta_out = jnp.where(new_done, theta, theta - ratio)
        return (theta_out, i + 1, new_done)

    def cond(carry):
        _, i, done = carry
        return jnp.logical_and(i < MAX_ITER, jnp.logical_not(done))

    theta, _, _ = lax.while_loop(
        cond, body, (theta0, jnp.int32(0), jnp.bool_(False))
    )
    return _final_call(s_flat, theta).reshape(N_ROWS, N_COLS)


# speedup vs baseline: 5.2122x; 5.2122x over previous
import jax
import jax.numpy as jnp
from jax.experimental import pallas as pl
from jax.experimental.pallas import tpu as pltpu

K = 2
TAU = 0.01
MAX_ITER = 100
TOL = 1e-3

N_ROWS = 16384
SUB = 8
LANE = N_ROWS // SUB


def _select_third_largest(c):
    v0 = jnp.minimum(c[0], c[1]); v1 = jnp.maximum(c[0], c[1])
    v2 = jnp.minimum(c[2], c[3]); v3 = jnp.maximum(c[2], c[3])
    v4 = jnp.minimum(c[4], c[5]); v5 = jnp.maximum(c[4], c[5])
    v6 = jnp.minimum(c[6], c[7]); v7 = jnp.maximum(c[6], c[7])
    w2 = jnp.maximum(v0, v2)
    w1 = jnp.minimum(v1, v3); w3 = jnp.maximum(v1, v3)
    w6 = jnp.maximum(v4, v6)
    w5 = jnp.minimum(v5, v7); w7 = jnp.maximum(v5, v7)
    x1 = jnp.minimum(w1, w2); x2 = jnp.maximum(w1, w2)
    x5 = jnp.minimum(w5, w6); x6 = jnp.maximum(w5, w6)
    y5 = jnp.maximum(x1, x5)
    y6 = jnp.maximum(x2, x6)
    y3 = jnp.minimum(w3, w7)
    z5 = jnp.maximum(y3, y5)
    return jnp.minimum(z5, y6)


def _gate_kernel(v_ref, g_ref):
    vt = v_ref[...].T
    st = (vt.reshape(16, 8, LANE // 2)
            .transpose(1, 0, 2)
            .reshape(8, SUB, LANE))

    theta0 = _select_third_largest([st[j] for j in range(8)])

    def body(carry):
        theta, i, done = carry
        sig = jax.nn.sigmoid((st - theta[None]) / TAU)
        f = jnp.sum(sig, axis=0) - K
        new_done = (jnp.sum(f) / N_ROWS) < TOL
        df = -(1.0 / TAU) * jnp.sum(sig * (1.0 - sig), axis=0)
        theta_new = theta - f / df
        theta_out = jnp.where(new_done, theta, theta_new)
        return (theta_out, i + 1, new_done)

    def cond(carry):
        _, i, done = carry
        return jnp.logical_and(i < MAX_ITER, jnp.logical_not(done))

    theta, _, _ = jax.lax.while_loop(
        cond, body, (theta0, jnp.int32(0), jnp.bool_(False))
    )

    g = jax.nn.sigmoid((st - theta[None]) / TAU)
    gt = (g.reshape(8, 16, LANE // 2)
           .transpose(1, 0, 2)
           .reshape(128, LANE // 2))
    g_ref[...] = gt.T


@jax.jit
def kernel(s):
    v = s.reshape(1024, 128)
    g_v = pl.pallas_call(
        _gate_kernel,
        out_shape=jax.ShapeDtypeStruct(v.shape, v.dtype),
        in_specs=[pl.BlockSpec(memory_space=pltpu.VMEM)],
        out_specs=pl.BlockSpec(memory_space=pltpu.VMEM),
    )(v)
    return g_v.reshape(N_ROWS, 8)


# speedup vs baseline: 39.0218x; 7.4867x over previous
import jax
import jax.numpy as jnp
from jax.experimental import pallas as pl
from jax.experimental.pallas import tpu as pltpu

K = 2
TAU = 0.01
MAX_ITER = 100
TOL = 1e-3

N_ROWS = 16384
SUB = 8
LANE = N_ROWS // SUB


def _sigmoid(x):
    return 0.5 * jnp.tanh(0.5 * x) + 0.5


def _select_third_largest(c):
    v0 = jnp.minimum(c[0], c[1]); v1 = jnp.maximum(c[0], c[1])
    v2 = jnp.minimum(c[2], c[3]); v3 = jnp.maximum(c[2], c[3])
    v4 = jnp.minimum(c[4], c[5]); v5 = jnp.maximum(c[4], c[5])
    v6 = jnp.minimum(c[6], c[7]); v7 = jnp.maximum(c[6], c[7])
    w2 = jnp.maximum(v0, v2)
    w1 = jnp.minimum(v1, v3); w3 = jnp.maximum(v1, v3)
    w6 = jnp.maximum(v4, v6)
    w5 = jnp.minimum(v5, v7); w7 = jnp.maximum(v5, v7)
    x1 = jnp.minimum(w1, w2); x2 = jnp.maximum(w1, w2)
    x5 = jnp.minimum(w5, w6); x6 = jnp.maximum(w5, w6)
    y5 = jnp.maximum(x1, x5)
    y6 = jnp.maximum(x2, x6)
    y3 = jnp.minimum(w3, w7)
    z5 = jnp.maximum(y3, y5)
    return jnp.minimum(z5, y6)


def _gate_kernel(st_ref, g_ref):
    st = st_ref[...].reshape(8, SUB, LANE)

    theta0 = _select_third_largest([st[j] for j in range(8)])

    sts = st / TAU
    t0 = theta0 / TAU

    def body(carry):
        theta, i, done = carry
        sig = _sigmoid(sts - theta[None])
        f = jnp.sum(sig, axis=0) - K
        new_done = (jnp.sum(f) / N_ROWS) < TOL
        w = jnp.sum(sig * (1.0 - sig), axis=0)
        theta_new = theta + f / w
        theta_out = jnp.where(new_done, theta, theta_new)
        return (theta_out, i + 1, new_done)

    def cond(carry):
        _, i, done = carry
        return jnp.logical_and(i < MAX_ITER, jnp.logical_not(done))

    theta, _, _ = jax.lax.while_loop(
        cond, body, (t0, jnp.int32(0), jnp.bool_(False))
    )

    g = _sigmoid(sts - theta[None])
    g_ref[...] = g.reshape(8, N_ROWS)


@jax.jit
def kernel(s):
    st = s.T
    g_t = pl.pallas_call(
        _gate_kernel,
        out_shape=jax.ShapeDtypeStruct(st.shape, st.dtype),
        in_specs=[pl.BlockSpec(memory_space=pltpu.VMEM)],
        out_specs=pl.BlockSpec(memory_space=pltpu.VMEM),
    )(st)
    return g_t.T
